# baseline (speedup 1.0000x reference)
import jax
import jax.numpy as jnp
import numpy as np
from jax.experimental import pallas as pl

_E = 8
_R = 16
_SCALING = 32.0 / 16.0
_BT = 512


def _smoe_block(x_ref, wg_ref, acat_ref, bcat_ref, srep_ref, o_ref):
    x = x_ref[...]
    logits = jnp.dot(x, wg_ref[...], preferred_element_type=jnp.float32)
    m1 = jnp.max(logits, axis=-1, keepdims=True)
    masked = jnp.where(logits == m1, -jnp.inf, logits)
    m2 = jnp.max(masked, axis=-1, keepdims=True)
    w1 = jax.nn.sigmoid(m1 - m2)
    gates = jnp.where(logits == m1, w1, jnp.where(logits == m2, 1.0 - w1, 0.0))
    gate_exp = jnp.dot(gates, srep_ref[...], preferred_element_type=jnp.float32)

    xb = x.astype(jnp.bfloat16)
    h = jnp.dot(xb, acat_ref[...], preferred_element_type=jnp.float32)
    wh = (h * gate_exp).astype(jnp.bfloat16)
    out = jnp.dot(wh, bcat_ref[...], preferred_element_type=jnp.float32)
    o_ref[...] = out * _SCALING


def kernel(inputs, patch_h, patch_w, Wg, A, Bw):
    orig_shape = inputs.shape
    D = orig_shape[-1]
    flat = inputs.reshape(-1, D)
    T = flat.shape[0]

    E, _, R = A.shape
    acat = A.transpose(1, 0, 2).reshape(D, E * R).astype(jnp.bfloat16)
    bcat = Bw.reshape(E * R, D).astype(jnp.bfloat16)
    srep = jnp.asarray(np.repeat(np.eye(E, dtype=np.float32), R, axis=1))

    grid = (T // _BT,)
    out = pl.pallas_call(
        _smoe_block,
        grid=grid,
        in_specs=[
            pl.BlockSpec((_BT, D), lambda i: (i, 0)),
            pl.BlockSpec((D, E), lambda i: (0, 0)),
            pl.BlockSpec((D, E * R), lambda i: (0, 0)),
            pl.BlockSpec((E * R, D), lambda i: (0, 0)),
            pl.BlockSpec((E, E * R), lambda i: (0, 0)),
        ],
        out_specs=pl.BlockSpec((_BT, D), lambda i: (i, 0)),
        out_shape=jax.ShapeDtypeStruct((T, D), jnp.float32),
    )(flat, Wg, acat, bcat, srep)
    return out.reshape(orig_shape[:-1] + (D,))


# speedup vs baseline: 1.1523x; 1.1523x over previous
import jax
import jax.numpy as jnp
import numpy as np
from jax.experimental import pallas as pl

_E = 8
_R = 16
_SCALING = 32.0 / 16.0
_BT = 2048


def _smoe_block(x_ref, wg_ref, acat_ref, bcat_ref, srep_ref, o_ref):
    x = x_ref[...]
    logits = jnp.dot(x, wg_ref[...], preferred_element_type=jnp.float32)
    m1 = jnp.max(logits, axis=-1, keepdims=True)
    masked = jnp.where(logits == m1, -jnp.inf, logits)
    m2 = jnp.max(masked, axis=-1, keepdims=True)
    w1 = jax.nn.sigmoid(m1 - m2)
    gates = jnp.where(logits == m1, w1, jnp.where(logits == m2, 1.0 - w1, 0.0))
    gate_exp = jnp.dot(gates, srep_ref[...], preferred_element_type=jnp.float32)

    xb = x.astype(jnp.bfloat16)
    h = jnp.dot(xb, acat_ref[...], preferred_element_type=jnp.float32)
    wh = (h * gate_exp).astype(jnp.bfloat16)
    out = jnp.dot(wh, bcat_ref[...], preferred_element_type=jnp.float32)
    o_ref[...] = out * _SCALING


def kernel(inputs, patch_h, patch_w, Wg, A, Bw):
    orig_shape = inputs.shape
    D = orig_shape[-1]
    flat = inputs.reshape(-1, D)
    T = flat.shape[0]

    E, _, R = A.shape
    acat = A.transpose(1, 0, 2).reshape(D, E * R).astype(jnp.bfloat16)
    bcat = Bw.reshape(E * R, D).astype(jnp.bfloat16)
    srep = jnp.asarray(np.repeat(np.eye(E, dtype=np.float32), R, axis=1))

    grid = (T // _BT,)
    out = pl.pallas_call(
        _smoe_block,
        grid=grid,
        in_specs=[
            pl.BlockSpec((_BT, D), lambda i: (i, 0)),
            pl.BlockSpec((D, E), lambda i: (0, 0)),
            pl.BlockSpec((D, E * R), lambda i: (0, 0)),
            pl.BlockSpec((E * R, D), lambda i: (0, 0)),
            pl.BlockSpec((E, E * R), lambda i: (0, 0)),
        ],
        out_specs=pl.BlockSpec((_BT, D), lambda i: (i, 0)),
        out_shape=jax.ShapeDtypeStruct((T, D), jnp.float32),
    )(flat, Wg, acat, bcat, srep)
    return out.reshape(orig_shape[:-1] + (D,))
